# baseline (speedup 1.0000x reference)
import functools

import jax
import jax.numpy as jnp
from jax import lax
from jax.experimental import pallas as pl
from jax.experimental.pallas import tpu as pltpu
from jax.experimental.pallas import tpu_sc as plsc

VOCAB = 1000000
EMB = 64
SEQ = 50
BATCH = 4096
HID = 64
DENSE = 512

_NC, _NS = 2, 16
_NW = _NC * _NS
_ROWS = BATCH * SEQ
_RPW = _ROWS // _NW
_CHUNK = 800
_NCH = _RPW // _CHUNK


@functools.cache
def _make_sc_gather():
    @functools.partial(
        pl.kernel,
        out_type=jax.ShapeDtypeStruct((_ROWS, EMB), jnp.float32),
        mesh=plsc.VectorSubcoreMesh(core_axis_name="c", subcore_axis_name="s"),
        scratch_types=[
            pltpu.VMEM((_RPW,), jnp.int32),
            pltpu.VMEM((_CHUNK, EMB), jnp.float32),
            pltpu.VMEM((_CHUNK, EMB), jnp.float32),
            pltpu.SemaphoreType.DMA,
            pltpu.SemaphoreType.DMA,
        ],
        compiler_params=pltpu.CompilerParams(use_tc_tiling_on_sc=False),
    )
    def sc_gather(idx_hbm, table_hbm, out_hbm, idx_v, buf0, buf1, sem0, sem1):
        wid = lax.axis_index("s") * _NC + lax.axis_index("c")
        base = wid * _RPW
        pltpu.sync_copy(idx_hbm.at[pl.ds(base, _RPW)], idx_v)
        bufs = (buf0, buf1)
        sems = (sem0, sem1)
        cp = pltpu.async_copy(table_hbm.at[idx_v.at[pl.ds(0, _CHUNK)]],
                              bufs[0], sems[0])
        for c in range(_NCH):
            nxt = c + 1
            cpn = None
            if nxt < _NCH:
                cpn = pltpu.async_copy(
                    table_hbm.at[idx_v.at[pl.ds(nxt * _CHUNK, _CHUNK)]],
                    bufs[nxt % 2], sems[nxt % 2])
            cp.wait()
            pltpu.sync_copy(bufs[c % 2],
                            out_hbm.at[pl.ds(base + c * _CHUNK, _CHUNK)])
            cp = cpn

    return sc_gather


_BB = 256
_GRID = BATCH // _BB


def _lstm_body(e_ref, x_ref, wx_ref, wh_ref, b_ref, wd_ref, bd_ref,
               out_ref, xz_ref):
    e = e_ref[...].reshape(SEQ * _BB, EMB)
    xz = jnp.dot(e, wx_ref[...], preferred_element_type=jnp.float32)
    xz_ref[...] = (xz + b_ref[...]).reshape(SEQ, _BB, 4 * HID)

    h = jnp.zeros((_BB, HID), jnp.float32)
    c = jnp.zeros((_BB, HID), jnp.float32)
    wh = wh_ref[...]
    for t in range(SEQ):
        z = xz_ref[t] + jnp.dot(h, wh, preferred_element_type=jnp.float32)
        i = jax.nn.sigmoid(z[:, 0:HID])
        f = jax.nn.sigmoid(z[:, HID:2 * HID])
        g = jnp.maximum(z[:, 2 * HID:3 * HID], 0.0)
        o = jax.nn.sigmoid(z[:, 3 * HID:4 * HID])
        c_new = f * c + i * g
        h_new = o * jnp.maximum(c_new, 0.0)
        m = x_ref[:, t:t + 1] != 0
        h = jnp.where(m, h_new, h)
        c = jnp.where(m, c_new, c)
    out_ref[...] = jnp.maximum(
        jnp.dot(h, wd_ref[...], preferred_element_type=jnp.float32)
        + bd_ref[...], 0.0)


def _lstm_call(e_tm, x, Wx, Wh, b2, Wd, bd2, interpret=False):
    return pl.pallas_call(
        _lstm_body,
        grid=(_GRID,),
        in_specs=[
            pl.BlockSpec((SEQ, _BB, EMB), lambda j: (0, j, 0)),
            pl.BlockSpec((_BB, SEQ), lambda j: (j, 0)),
            pl.BlockSpec((EMB, 4 * HID), lambda j: (0, 0)),
            pl.BlockSpec((HID, 4 * HID), lambda j: (0, 0)),
            pl.BlockSpec((1, 4 * HID), lambda j: (0, 0)),
            pl.BlockSpec((HID, DENSE), lambda j: (0, 0)),
            pl.BlockSpec((1, DENSE), lambda j: (0, 0)),
        ],
        out_specs=pl.BlockSpec((_BB, DENSE), lambda j: (j, 0)),
        out_shape=jax.ShapeDtypeStruct((BATCH, DENSE), jnp.float32),
        scratch_shapes=[pltpu.VMEM((SEQ, _BB, 4 * HID), jnp.float32)],
        compiler_params=pltpu.CompilerParams(
            dimension_semantics=("arbitrary",)),
        interpret=interpret,
    )(e_tm, x, Wx, Wh, b2, Wd, bd2)


def kernel(x, emb_table, Wx, Wh, b, Wd, bd):
    idx = jnp.transpose(x).reshape(-1)
    e = _make_sc_gather()(idx, emb_table)
    e_tm = e.reshape(SEQ, BATCH, EMB)
    return _lstm_call(e_tm, x, Wx, Wh, b.reshape(1, -1), Wd, bd.reshape(1, -1))


# speedup vs baseline: 1.1619x; 1.1619x over previous
import functools

import jax
import jax.numpy as jnp
from jax import lax
from jax.experimental import pallas as pl
from jax.experimental.pallas import tpu as pltpu
from jax.experimental.pallas import tpu_sc as plsc

VOCAB = 1000000
EMB = 64
SEQ = 50
BATCH = 4096
HID = 64
DENSE = 512

_NC, _NS = 2, 16
_NW = _NC * _NS
_ROWS = BATCH * SEQ
_RPW = _ROWS // _NW
_CHUNK = 800
_NCH = _RPW // _CHUNK


@functools.cache
def _make_sc_gather():
    @functools.partial(
        pl.kernel,
        out_type=jax.ShapeDtypeStruct((_ROWS, EMB), jnp.float32),
        mesh=plsc.VectorSubcoreMesh(core_axis_name="c", subcore_axis_name="s"),
        scratch_types=[
            pltpu.VMEM((_RPW,), jnp.int32),
            pltpu.VMEM((_CHUNK, EMB), jnp.float32),
            pltpu.VMEM((_CHUNK, EMB), jnp.float32),
            pltpu.SemaphoreType.DMA,
            pltpu.SemaphoreType.DMA,
        ],
        compiler_params=pltpu.CompilerParams(use_tc_tiling_on_sc=False),
    )
    def sc_gather(idx_hbm, table_hbm, out_hbm, idx_v, buf0, buf1, sem0, sem1):
        wid = lax.axis_index("s") * _NC + lax.axis_index("c")
        base = wid * _RPW
        pltpu.sync_copy(idx_hbm.at[pl.ds(base, _RPW)], idx_v)
        bufs = (buf0, buf1)
        sems = (sem0, sem1)
        cp = pltpu.async_copy(table_hbm.at[idx_v.at[pl.ds(0, _CHUNK)]],
                              bufs[0], sems[0])
        for c in range(_NCH):
            nxt = c + 1
            cpn = None
            if nxt < _NCH:
                cpn = pltpu.async_copy(
                    table_hbm.at[idx_v.at[pl.ds(nxt * _CHUNK, _CHUNK)]],
                    bufs[nxt % 2], sems[nxt % 2])
            cp.wait()
            pltpu.sync_copy(bufs[c % 2],
                            out_hbm.at[pl.ds(base + c * _CHUNK, _CHUNK)])
            cp = cpn

    return sc_gather


_BB = 512
_GRID = BATCH // _BB


def _sigmoid(x):
    return 0.5 * jnp.tanh(0.5 * x) + 0.5


def _lstm_body(e_ref, x_ref, wx_ref, whp_ref, b_ref, wdp_ref, bd_ref,
               out_ref, xz_ref):
    e = e_ref[...].reshape(SEQ * _BB, EMB)
    xz = jnp.dot(e, wx_ref[...], preferred_element_type=jnp.float32)
    xz_ref[...] = (xz + b_ref[...]).reshape(SEQ, _BB, 4 * HID)

    lanemask = jax.lax.broadcasted_iota(jnp.int32, (_BB, 2 * HID), 1) < HID
    h128 = jnp.zeros((_BB, 2 * HID), jnp.float32)
    c128 = jnp.zeros((_BB, 2 * HID), jnp.float32)
    whp = whp_ref[...]
    for t in range(SEQ):
        z = xz_ref[t] + jnp.dot(h128, whp, preferred_element_type=jnp.float32)
        z0 = z[:, 0:2 * HID]
        z1 = z[:, 2 * HID:4 * HID]
        s = _sigmoid(z0)
        a = jnp.where(lanemask, jnp.maximum(z1, 0.0), _sigmoid(z1))
        p = s * a
        q = s * c128
        cn = q + pltpu.roll(p, HID, 1)
        hn = a * jnp.maximum(cn, 0.0)
        m = x_ref[:, t:t + 1] != 0
        c128 = jnp.where(m, cn, c128)
        h128 = jnp.where(m, hn, h128)
    out_ref[...] = jnp.maximum(
        jnp.dot(h128, wdp_ref[...], preferred_element_type=jnp.float32)
        + bd_ref[...], 0.0)


def _lstm_call(e_tm, x, Wx, Whp, b2, Wdp, bd2, interpret=False):
    return pl.pallas_call(
        _lstm_body,
        grid=(_GRID,),
        in_specs=[
            pl.BlockSpec((SEQ, _BB, EMB), lambda j: (0, j, 0)),
            pl.BlockSpec((_BB, SEQ), lambda j: (j, 0)),
            pl.BlockSpec((EMB, 4 * HID), lambda j: (0, 0)),
            pl.BlockSpec((2 * HID, 4 * HID), lambda j: (0, 0)),
            pl.BlockSpec((1, 4 * HID), lambda j: (0, 0)),
            pl.BlockSpec((2 * HID, DENSE), lambda j: (0, 0)),
            pl.BlockSpec((1, DENSE), lambda j: (0, 0)),
        ],
        out_specs=pl.BlockSpec((_BB, DENSE), lambda j: (j, 0)),
        out_shape=jax.ShapeDtypeStruct((BATCH, DENSE), jnp.float32),
        scratch_shapes=[pltpu.VMEM((SEQ, _BB, 4 * HID), jnp.float32)],
        compiler_params=pltpu.CompilerParams(
            dimension_semantics=("arbitrary",),
            vmem_limit_bytes=120 * 1024 * 1024),
        interpret=interpret,
    )(e_tm, x, Wx, Whp, b2, Wdp, bd2)


def kernel(x, emb_table, Wx, Wh, b, Wd, bd):
    idx = jnp.transpose(x).reshape(-1)
    e = _make_sc_gather()(idx, emb_table)
    e_tm = e.reshape(SEQ, BATCH, EMB)
    Whp = jnp.concatenate([jnp.zeros((HID, 4 * HID), jnp.float32), Wh], axis=0)
    Wdp = jnp.concatenate([jnp.zeros((HID, DENSE), jnp.float32), Wd], axis=0)
    return _lstm_call(e_tm, x, Wx, Whp, b.reshape(1, -1), Wdp,
                      bd.reshape(1, -1))


# speedup vs baseline: 1.2892x; 1.1095x over previous
import functools

import jax
import jax.numpy as jnp
from jax import lax
from jax.experimental import pallas as pl
from jax.experimental.pallas import tpu as pltpu
from jax.experimental.pallas import tpu_sc as plsc

VOCAB = 1000000
EMB = 64
SEQ = 50
BATCH = 4096
HID = 64
DENSE = 512

_NC, _NS = 2, 16
_NW = _NC * _NS
_ROWS = BATCH * SEQ
_RPW = _ROWS // _NW
_CHUNK = 800
_NCH = _RPW // _CHUNK


@functools.cache
def _make_sc_gather():
    @functools.partial(
        pl.kernel,
        out_type=jax.ShapeDtypeStruct((_ROWS, 2 * EMB), jnp.float32),
        mesh=plsc.VectorSubcoreMesh(core_axis_name="c", subcore_axis_name="s"),
        scratch_types=[
            pltpu.VMEM((_RPW,), jnp.int32),
            pltpu.VMEM((_CHUNK, EMB), jnp.float32),
            pltpu.VMEM((_CHUNK, EMB), jnp.float32),
            pltpu.SemaphoreType.DMA,
            pltpu.SemaphoreType.DMA,
        ],
        compiler_params=pltpu.CompilerParams(use_tc_tiling_on_sc=False),
    )
    def sc_gather(idx_hbm, table_hbm, out_hbm, idx_v, buf0, buf1, sem0, sem1):
        wid = lax.axis_index("s") * _NC + lax.axis_index("c")
        base = wid * _RPW
        pltpu.sync_copy(idx_hbm.at[pl.ds(base, _RPW)], idx_v)
        bufs = (buf0, buf1)
        sems = (sem0, sem1)
        cp = pltpu.async_copy(table_hbm.at[idx_v.at[pl.ds(0, _CHUNK)]],
                              bufs[0], sems[0])
        for c in range(_NCH):
            nxt = c + 1
            cpn = None
            if nxt < _NCH:
                cpn = pltpu.async_copy(
                    table_hbm.at[idx_v.at[pl.ds(nxt * _CHUNK, _CHUNK)]],
                    bufs[nxt % 2], sems[nxt % 2])
            cp.wait()
            pltpu.sync_copy(
                bufs[c % 2],
                out_hbm.at[pl.ds(base + c * _CHUNK, _CHUNK), pl.ds(0, EMB)])
            cp = cpn

    return sc_gather


_BB = 512
_GRID = BATCH // _BB


def _sigmoid(x):
    return 0.5 * jnp.tanh(0.5 * x) + 0.5


def _lstm_body(e_ref, x_ref, w_ref, b_ref, wdp_ref, bd_ref, out_ref):
    lanemask = jax.lax.broadcasted_iota(jnp.int32, (_BB, 2 * HID), 1) < HID
    h128 = jnp.zeros((_BB, 2 * HID), jnp.float32)
    c128 = jnp.zeros((_BB, 2 * HID), jnp.float32)
    w = w_ref[...]
    b = b_ref[...]
    for t in range(SEQ):
        u = jnp.where(lanemask, e_ref[t], h128)
        z = jnp.dot(u, w, preferred_element_type=jnp.float32) + b
        z0 = z[:, 0:2 * HID]
        z1 = z[:, 2 * HID:4 * HID]
        s = _sigmoid(z0)
        a = jnp.where(lanemask, jnp.maximum(z1, 0.0), _sigmoid(z1))
        p = s * a
        q = s * c128
        cn = q + pltpu.roll(p, HID, 1)
        hn = a * jnp.maximum(cn, 0.0)
        m = x_ref[:, t:t + 1] != 0
        c128 = jnp.where(m, cn, c128)
        h128 = jnp.where(m, hn, h128)
    out_ref[...] = jnp.maximum(
        jnp.dot(h128, wdp_ref[...], preferred_element_type=jnp.float32)
        + bd_ref[...], 0.0)


def _lstm_call(e3, x, W, b2, Wdp, bd2, interpret=False):
    return pl.pallas_call(
        _lstm_body,
        grid=(_GRID,),
        in_specs=[
            pl.BlockSpec((SEQ, _BB, 2 * EMB), lambda j: (0, j, 0)),
            pl.BlockSpec((_BB, SEQ), lambda j: (j, 0)),
            pl.BlockSpec((2 * HID, 4 * HID), lambda j: (0, 0)),
            pl.BlockSpec((1, 4 * HID), lambda j: (0, 0)),
            pl.BlockSpec((2 * HID, DENSE), lambda j: (0, 0)),
            pl.BlockSpec((1, DENSE), lambda j: (0, 0)),
        ],
        out_specs=pl.BlockSpec((_BB, DENSE), lambda j: (j, 0)),
        out_shape=jax.ShapeDtypeStruct((BATCH, DENSE), jnp.float32),
        compiler_params=pltpu.CompilerParams(
            dimension_semantics=("arbitrary",),
            vmem_limit_bytes=120 * 1024 * 1024),
        interpret=interpret,
    )(e3, x, W, b2, Wdp, bd2)


def kernel(x, emb_table, Wx, Wh, b, Wd, bd):
    idx = jnp.transpose(x).reshape(-1)
    e128 = _make_sc_gather()(idx, emb_table)
    e3 = e128.reshape(SEQ, BATCH, 2 * EMB)
    W = jnp.concatenate([Wx, Wh], axis=0)
    Wdp = jnp.concatenate([jnp.zeros((HID, DENSE), jnp.float32), Wd], axis=0)
    return _lstm_call(e3, x, W, b.reshape(1, -1), Wdp, bd.reshape(1, -1))
